# baseline (speedup 1.0000x reference)
import functools

import jax
import jax.numpy as jnp
from jax.experimental import pallas as pl

_K = 32
_LANES = 128
_TOPJ = 5
_BISECT_ITERS = 21
_BLOCK_ROWS = 200
_SUB_ROWS = 40


def _topk_mask_body(a_ref, o_ref, *, k, iters):
    rblk, n = a_ref.shape
    L = _LANES
    nf = n // L
    rem = n - nf * L
    t = _SUB_ROWS if rblk % _SUB_ROWS == 0 else rblk
    neg = jnp.asarray(-jnp.inf, a_ref.dtype)

    def subtile(s, carry):
        rows = pl.ds(s * t, t)
        nseed = min(_TOPJ, nf)
        S = [a_ref[rows, c * L:(c + 1) * L] for c in range(nseed)]
        S += [jnp.full((t, L), neg, a_ref.dtype) for _ in range(_TOPJ - nseed)]

        def insert(v):
            for j in range(_TOPJ):
                top = jnp.maximum(S[j], v)
                if j < _TOPJ - 1:
                    v = jnp.minimum(S[j], v)
                S[j] = top

        for c in range(nseed, nf):
            insert(a_ref[rows, c * L:(c + 1) * L])
        if rem:
            tail = a_ref[rows, nf * L:n]
            pad = jnp.full((t, L - rem), neg, a_ref.dtype)
            insert(jnp.concatenate([tail, pad], axis=1))

        cand = jnp.concatenate(S, axis=1)
        cmax = jnp.max(cand, axis=1, keepdims=True)
        hi = jnp.maximum(cmax, 0.0) * (1.0 + 1e-4) + 1e-20
        lo = jnp.zeros_like(hi)

        def step(_, bracket):
            lo, hi = bracket
            m = 0.5 * (lo + hi)
            c = jnp.sum(jnp.where(cand >= m, 1.0, 0.0), axis=1, keepdims=True)
            ge = c >= k
            return jnp.where(ge, m, lo), jnp.where(ge, hi, m)

        lo, hi = jax.lax.fori_loop(0, iters, step, (lo, hi))
        av = a_ref[rows, :]
        o_ref[rows, :] = jnp.where(av >= lo, av, 0.0)
        return carry

    jax.lax.fori_loop(0, rblk // t, subtile, 0)


def kernel(idx, A):
    del idx
    n, m = A.shape
    block_rows = _BLOCK_ROWS if n % _BLOCK_ROWS == 0 else n
    grid = (n // block_rows,)
    body = functools.partial(_topk_mask_body, k=_K, iters=_BISECT_ITERS)
    return pl.pallas_call(
        body,
        grid=grid,
        in_specs=[pl.BlockSpec((block_rows, m), lambda i: (i, 0))],
        out_specs=pl.BlockSpec((block_rows, m), lambda i: (i, 0)),
        out_shape=jax.ShapeDtypeStruct((n, m), A.dtype),
    )(A)


# speedup vs baseline: 2.0993x; 2.0993x over previous
import functools

import jax
import jax.numpy as jnp
from jax.experimental import pallas as pl

_K = 32
_LANES = 128
_TOPJ = 5
_BISECT_ITERS = 21
_BLOCK_ROWS = 200
_SUB_ROWS = 40


def _lane_topj(a_ref, r0, t, nf, rem, n):
    L = _LANES
    neg = jnp.asarray(-jnp.inf, a_ref.dtype)
    nseed = min(_TOPJ, nf)
    S = [a_ref[r0:r0 + t, c * L:(c + 1) * L] for c in range(nseed)]
    S += [jnp.full((t, L), neg, a_ref.dtype) for _ in range(_TOPJ - nseed)]

    def insert(v):
        for j in range(_TOPJ):
            top = jnp.maximum(S[j], v)
            if j < _TOPJ - 1:
                v = jnp.minimum(S[j], v)
            S[j] = top

    for c in range(nseed, nf):
        insert(a_ref[r0:r0 + t, c * L:(c + 1) * L])
    if rem:
        tail = a_ref[r0:r0 + t, nf * L:n]
        pad = jnp.full((t, L - rem), neg, a_ref.dtype)
        insert(jnp.concatenate([tail, pad], axis=1))
    return jnp.concatenate(S, axis=1)


def _topk_mask_body(a_ref, o_ref, *, k, iters):
    rblk, n = a_ref.shape
    L = _LANES
    nf = n // L
    rem = n - nf * L
    t = _SUB_ROWS if rblk % _SUB_ROWS == 0 else rblk

    cand = jnp.concatenate(
        [_lane_topj(a_ref, r0, t, nf, rem, n) for r0 in range(0, rblk, t)],
        axis=0)

    cmax = jnp.max(cand, axis=1, keepdims=True)
    hi = jnp.maximum(cmax, 0.0) * (1.0 + 1e-4) + 1e-20
    lo = jnp.zeros_like(hi)

    def step(_, bracket):
        lo, hi = bracket
        m = 0.5 * (lo + hi)
        c = jnp.sum(jnp.where(cand >= m, 1.0, 0.0), axis=1, keepdims=True)
        ge = c >= k
        return jnp.where(ge, m, lo), jnp.where(ge, hi, m)

    lo, hi = jax.lax.fori_loop(0, iters, step, (lo, hi))
    av = a_ref[...]
    o_ref[...] = jnp.where(av >= lo, av, 0.0)


def kernel(idx, A):
    del idx
    n, m = A.shape
    block_rows = _BLOCK_ROWS if n % _BLOCK_ROWS == 0 else n
    grid = (n // block_rows,)
    body = functools.partial(_topk_mask_body, k=_K, iters=_BISECT_ITERS)
    return pl.pallas_call(
        body,
        grid=grid,
        in_specs=[pl.BlockSpec((block_rows, m), lambda i: (i, 0))],
        out_specs=pl.BlockSpec((block_rows, m), lambda i: (i, 0)),
        out_shape=jax.ShapeDtypeStruct((n, m), A.dtype),
    )(A)


# speedup vs baseline: 2.3122x; 1.1014x over previous
import functools

import jax
import jax.numpy as jnp
from jax.experimental import pallas as pl

_K = 32
_LANES = 128
_TOPJ = 4
_BISECT_ITERS = 18
_BLOCK_ROWS = 200
_SUB_ROWS = 40


def _lane_topj(a_ref, r0, t, nf, rem, n):
    L = _LANES
    neg = jnp.asarray(-jnp.inf, a_ref.dtype)
    nseed = min(_TOPJ, nf)
    S = [a_ref[r0:r0 + t, c * L:(c + 1) * L] for c in range(nseed)]
    S += [jnp.full((t, L), neg, a_ref.dtype) for _ in range(_TOPJ - nseed)]

    def insert(v):
        for j in range(_TOPJ):
            top = jnp.maximum(S[j], v)
            if j < _TOPJ - 1:
                v = jnp.minimum(S[j], v)
            S[j] = top

    for c in range(nseed, nf):
        insert(a_ref[r0:r0 + t, c * L:(c + 1) * L])
    if rem:
        tail = a_ref[r0:r0 + t, nf * L:n]
        pad = jnp.full((t, L - rem), neg, a_ref.dtype)
        insert(jnp.concatenate([tail, pad], axis=1))
    return jnp.concatenate(S, axis=1)


def _topk_mask_body(a_ref, o_ref, *, k, iters):
    rblk, n = a_ref.shape
    L = _LANES
    nf = n // L
    rem = n - nf * L
    t = _SUB_ROWS if rblk % _SUB_ROWS == 0 else rblk

    cand = jnp.concatenate(
        [_lane_topj(a_ref, r0, t, nf, rem, n) for r0 in range(0, rblk, t)],
        axis=0)

    cmax = jnp.max(cand, axis=1, keepdims=True)
    hi = jnp.maximum(cmax, 0.0) * (1.0 + 1e-4) + 1e-20
    lo = jnp.zeros_like(hi)

    def step(_, bracket):
        lo, hi = bracket
        m = 0.5 * (lo + hi)
        c = jnp.sum(jnp.where(cand >= m, 1.0, 0.0), axis=1, keepdims=True)
        ge = c >= k
        return jnp.where(ge, m, lo), jnp.where(ge, hi, m)

    lo, hi = jax.lax.fori_loop(0, iters, step, (lo, hi))
    av = a_ref[...]
    o_ref[...] = jnp.where(av >= lo, av, 0.0)


def kernel(idx, A):
    del idx
    n, m = A.shape
    block_rows = _BLOCK_ROWS if n % _BLOCK_ROWS == 0 else n
    grid = (n // block_rows,)
    body = functools.partial(_topk_mask_body, k=_K, iters=_BISECT_ITERS)
    return pl.pallas_call(
        body,
        grid=grid,
        in_specs=[pl.BlockSpec((block_rows, m), lambda i: (i, 0))],
        out_specs=pl.BlockSpec((block_rows, m), lambda i: (i, 0)),
        out_shape=jax.ShapeDtypeStruct((n, m), A.dtype),
    )(A)
